# Initial kernel scaffold; baseline (speedup 1.0000x reference)
import jax
import jax.numpy as jnp
from jax.experimental import pallas as pl


def kernel(maze_grid, cell_table, pos_table):
    raise NotImplementedError("write your pallas kernel here")


# speedup vs baseline: 3.9074x; 3.9074x over previous
import functools

import jax
import jax.numpy as jnp
from jax import lax
from jax.experimental import pallas as pl
from jax.experimental.pallas import tpu as pltpu
from jax.experimental.pallas import tpu_sc as plsc

MAZE = 32
P = MAZE * MAZE
D = 64
V = 4
TBL = V * P
NC, NS, L = 2, 16, 16
NW = NC * NS
CH = 128


def _sc_encode(maze_flat, cell_table, pos_table, n_rows):
    per_w = n_rows // NW
    nchunks = per_w // CH
    rows_per_sub = TBL // NS

    mesh = plsc.VectorSubcoreMesh(core_axis_name="c", subcore_axis_name="s")

    @functools.partial(
        pl.kernel,
        out_type=jax.ShapeDtypeStruct((n_rows, D), jnp.float32),
        mesh=mesh,
        compiler_params=pltpu.CompilerParams(use_tc_tiling_on_sc=False),
        scratch_types=[
            pltpu.VMEM_SHARED((TBL, D), jnp.float32),
            pltpu.VMEM((rows_per_sub, D), jnp.float32),
            pltpu.VMEM((D,), jnp.float32),
            pltpu.VMEM((CH,), jnp.int32),
            pltpu.VMEM((CH,), jnp.int32),
            pltpu.VMEM((CH, D), jnp.float32),
            pltpu.SemaphoreType.DMA,
        ],
    )
    def k(maze_hbm, cell_hbm, pos_hbm, out_hbm,
          tbl_sh, bbuf, crow, gbuf, ibuf, obuf, sem):
        cid = lax.axis_index("c")
        sid = lax.axis_index("s")
        wid = sid * NC + cid

        row0 = sid * rows_per_sub
        v = row0 // P
        pbase = row0 % P
        pltpu.sync_copy(cell_hbm.at[v], crow)
        pltpu.sync_copy(pos_hbm.at[pl.ds(pbase, rows_per_sub)], bbuf)
        c0 = crow[pl.ds(0, L)]
        c1 = crow[pl.ds(L, L)]
        c2 = crow[pl.ds(2 * L, L)]
        c3 = crow[pl.ds(3 * L, L)]

        def add_row(r, _):
            bbuf[r, pl.ds(0, L)] += c0
            bbuf[r, pl.ds(L, L)] += c1
            bbuf[r, pl.ds(2 * L, L)] += c2
            bbuf[r, pl.ds(3 * L, L)] += c3
            return _

        lax.fori_loop(0, rows_per_sub, add_row, 0)
        pltpu.sync_copy(bbuf, tbl_sh.at[pl.ds(row0, rows_per_sub)])
        plsc.subcore_barrier()

        base0 = wid * per_w
        lanes = lax.iota(jnp.int32, L)

        def do_chunk(t, _):
            base = base0 + t * CH
            pb = lax.rem(t, P // CH) * CH
            pltpu.sync_copy(maze_hbm.at[pl.ds(base, CH)], gbuf)
            for j in range(CH // L):
                g = gbuf[pl.ds(j * L, L)]
                ibuf[pl.ds(j * L, L)] = g * P + (pb + j * L) + lanes
            pltpu.async_copy(tbl_sh.at[ibuf], obuf, sem).wait()
            pltpu.sync_copy(obuf, out_hbm.at[pl.ds(base, CH)])
            return _

        lax.fori_loop(0, nchunks, do_chunk, 0)

    return k(maze_flat, cell_table, pos_table)


def kernel(maze_grid, cell_table, pos_table):
    batch, h, w = maze_grid.shape
    n_rows = batch * h * w
    maze_flat = maze_grid.reshape(n_rows).astype(jnp.int32)
    out = _sc_encode(maze_flat, cell_table, pos_table, n_rows)
    return out.reshape(batch, h * w, D)


# speedup vs baseline: 5.1474x; 1.3173x over previous
import functools

import jax
import jax.numpy as jnp
from jax import lax
from jax.experimental import pallas as pl
from jax.experimental.pallas import tpu as pltpu
from jax.experimental.pallas import tpu_sc as plsc

MAZE = 32
P = MAZE * MAZE
D = 64
V = 4
TBL = V * P
NC, NS, L = 2, 16, 16
NW = NC * NS
CH = 256
NBUF = 4


def _sc_encode(maze_flat, cell_table, pos_table, n_rows):
    per_w = n_rows // NW
    nchunks = per_w // CH
    rows_per_sub = TBL // NS

    mesh = plsc.VectorSubcoreMesh(core_axis_name="c", subcore_axis_name="s")

    @functools.partial(
        pl.kernel,
        out_type=jax.ShapeDtypeStruct((n_rows, D), jnp.float32),
        mesh=mesh,
        compiler_params=pltpu.CompilerParams(use_tc_tiling_on_sc=False),
        scratch_types=[
            pltpu.VMEM_SHARED((TBL, D), jnp.float32),
            pltpu.VMEM((D,), jnp.float32),
            pltpu.VMEM((per_w,), jnp.int32),
        ] + [pltpu.VMEM((CH,), jnp.int32) for _ in range(NBUF)]
          + [pltpu.VMEM((CH, D), jnp.float32) for _ in range(NBUF)]
          + [pltpu.SemaphoreType.DMA for _ in range(2 * NBUF)],
    )
    def k(maze_hbm, cell_hbm, pos_hbm, out_hbm,
          tbl_sh, crow, gbuf, *ring):
        ibufs = ring[:NBUF]
        obufs = ring[NBUF:2 * NBUF]
        gsems = ring[2 * NBUF:3 * NBUF]
        osems = ring[3 * NBUF:4 * NBUF]
        bbuf = obufs[0]
        cid = lax.axis_index("c")
        sid = lax.axis_index("s")
        wid = sid * NC + cid

        row0 = sid * rows_per_sub
        v = row0 // P
        pbase = row0 % P
        pltpu.sync_copy(cell_hbm.at[v], crow)
        pltpu.sync_copy(pos_hbm.at[pl.ds(pbase, rows_per_sub)], bbuf)
        c0 = crow[pl.ds(0, L)]
        c1 = crow[pl.ds(L, L)]
        c2 = crow[pl.ds(2 * L, L)]
        c3 = crow[pl.ds(3 * L, L)]

        def add_row(r, _):
            bbuf[r, pl.ds(0, L)] += c0
            bbuf[r, pl.ds(L, L)] += c1
            bbuf[r, pl.ds(2 * L, L)] += c2
            bbuf[r, pl.ds(3 * L, L)] += c3
            return _

        lax.fori_loop(0, rows_per_sub, add_row, 0)
        pltpu.sync_copy(bbuf, tbl_sh.at[pl.ds(row0, rows_per_sub)])
        plsc.subcore_barrier()

        base0 = wid * per_w
        lanes = lax.iota(jnp.int32, L)
        pltpu.sync_copy(maze_hbm.at[pl.ds(base0, per_w)], gbuf)

        def issue_gather(t, b):
            pb = lax.rem(t, P // CH) * CH
            for j in range(CH // L):
                g = gbuf[pl.ds(t * CH + j * L, L)]
                ibufs[b][pl.ds(j * L, L)] = g * P + (pb + j * L) + lanes
            pltpu.async_copy(tbl_sh.at[ibufs[b]], obufs[b], gsems[b])

        def wait_gather(b):
            pltpu.make_async_copy(tbl_sh.at[ibufs[b]], obufs[b], gsems[b]).wait()

        def issue_out(t, b):
            pltpu.async_copy(obufs[b], out_hbm.at[pl.ds(base0 + t * CH, CH)],
                             osems[b])

        def wait_out(t, b):
            pltpu.make_async_copy(obufs[b],
                                  out_hbm.at[pl.ds(base0 + t * CH, CH)],
                                  osems[b]).wait()

        for b in range(NBUF):
            issue_gather(b, b)
            if b >= 1:
                wait_gather(b - 1)
                issue_out(b - 1, b - 1)

        def group(gi, _):
            for b in range(NBUF):
                t = gi * NBUF + b
                wait_out(t - NBUF, b)
                issue_gather(t, b)
                prev = (b - 1) % NBUF
                wait_gather(prev)
                issue_out(t - 1, prev)
            return _

        lax.fori_loop(1, nchunks // NBUF, group, 0)

        last = nchunks - 1
        wait_gather(last % NBUF)
        issue_out(last, last % NBUF)
        for b in range(NBUF):
            t = nchunks - NBUF + b
            wait_out(t, b)

    return k(maze_flat, cell_table, pos_table)


def kernel(maze_grid, cell_table, pos_table):
    batch, h, w = maze_grid.shape
    n_rows = batch * h * w
    maze_flat = maze_grid.reshape(n_rows).astype(jnp.int32)
    out = _sc_encode(maze_flat, cell_table, pos_table, n_rows)
    return out.reshape(batch, h * w, D)
